# baseline (speedup 1.0000x reference)
import functools

import jax
import jax.numpy as jnp
from jax import lax
from jax.experimental import pallas as pl
from jax.experimental.pallas import tpu as pltpu
from jax.experimental.pallas import tpu_sc as plsc

N = 10000
D = 256
HALF = 128
OUT_D = 128

NC = 2
NS = 16
LANES = 16

CHUNK = 128
NPHASES = 3
PHASE_N = 3584
N_PAD = NPHASES * PHASE_N
ACC_ROWS = 3592
ZROWS = PHASE_N // NS
OROWS = PHASE_N // NS
TRASH_DST = N


def _make_agg_kernel(e_pad: int, with_cnt: bool):
    ept = e_pad // NS
    iters = ept // CHUNK

    out_type = [jax.ShapeDtypeStruct((NC, N_PAD, HALF), jnp.float32)]
    if with_cnt:
        out_type.append(jax.ShapeDtypeStruct((N_PAD, HALF), jnp.float32))

    scratch_types = [
        pltpu.VMEM_SHARED((ACC_ROWS, HALF), jnp.float32),
        pltpu.VMEM((ZROWS, HALF), jnp.float32),
        pltpu.VMEM((CHUNK, HALF), jnp.float32),
        pltpu.VMEM((CHUNK,), jnp.int32),
        pltpu.VMEM((CHUNK,), jnp.int32),
        pltpu.VMEM((CHUNK,), jnp.int32),
        pltpu.VMEM((CHUNK,), jnp.int32),
        pltpu.VMEM((CHUNK, HALF), jnp.float32),
        pltpu.VMEM((CHUNK, HALF), jnp.float32),
        pltpu.SemaphoreType.DMA,
        pltpu.SemaphoreType.DMA,
        pltpu.SemaphoreType.DMA,
    ]

    mesh = plsc.VectorSubcoreMesh(core_axis_name="c", subcore_axis_name="s")

    @functools.partial(
        pl.kernel, mesh=mesh, out_type=out_type, scratch_types=scratch_types)
    def agg(*refs):
        if with_cnt:
            (xi, srcp, dstp, agg_out, cnt_out,
             acc, zbuf, onesb, srcbA, dstbA, srcbB, dstbB, rowsA, rowsB,
             semGA, semSA, semSB) = refs
        else:
            (xi, srcp, dstp, agg_out,
             acc, zbuf, onesb, srcbA, dstbA, srcbB, dstbB, rowsA, rowsB,
             semGA, semSA, semSB) = refs
            cnt_out = None

        c = lax.axis_index("c")
        s = lax.axis_index("s")
        ebase = s * ept

        zero16 = jnp.zeros((LANES,), jnp.float32)
        one16 = jnp.ones((LANES,), jnp.float32)

        def zrow(i, _):
            for j in range(HALF // LANES):
                zbuf[i, pl.ds(j * LANES, LANES)] = zero16
            return 0

        lax.fori_loop(0, ZROWS, zrow, 0)

        if with_cnt:
            def orow(i, _):
                for j in range(HALF // LANES):
                    onesb[i, pl.ds(j * LANES, LANES)] = one16
                return 0
            lax.fori_loop(0, CHUNK, orow, 0)

        zrows = pl.ds(pl.multiple_of(s * ZROWS, 8), ZROWS)
        orows = pl.ds(pl.multiple_of(s * OROWS, 8), OROWS)

        def xform_src(buf):
            for j in range(CHUNK // LANES):
                sl = pl.ds(j * LANES, LANES)
                buf[sl] = buf[sl] * 2 + c

        def remap_dst(buf, pbase):
            for j in range(CHUNK // LANES):
                sl = pl.ds(j * LANES, LANES)
                t = buf[sl] - pbase
                ok = jnp.logical_and(t >= 0, t < PHASE_N)
                buf[sl] = jnp.where(ok, t, PHASE_N)

        def off1(i):
            return pl.ds(pl.multiple_of(ebase + i * CHUNK, CHUNK), CHUNK)

        def phase_body(phase, _):
            pltpu.sync_copy(zbuf, acc.at[zrows])
            plsc.subcore_barrier()

            pbase = phase * PHASE_N

            def step(k, _):
                dsA = off1(2 * k)
                pltpu.sync_copy(srcp.at[dsA], srcbA)
                pltpu.sync_copy(dstp.at[dsA], dstbA)
                for j in range(CHUNK // LANES):
                    sl = pl.ds(j * LANES, LANES)
                    srcbA[sl] = srcbA[sl] * 2 + c
                    t = dstbA[sl] - pbase
                    ok = jnp.logical_and(t >= 0, t < PHASE_N)
                    dstbA[sl] = jnp.where(ok, t, PHASE_N)
                pltpu.async_copy(xi.at[srcbA], rowsA, semGA).wait()
                sa = pltpu.async_copy(rowsA, acc.at[dstbA], semSA, add=True)
                dsB = off1(2 * k + 1)
                pltpu.sync_copy(srcp.at[dsB], srcbB)
                pltpu.sync_copy(dstp.at[dsB], dstbB)
                for j in range(CHUNK // LANES):
                    sl = pl.ds(j * LANES, LANES)
                    srcbB[sl] = srcbB[sl] * 2 + c
                    t = dstbB[sl] - pbase
                    ok = jnp.logical_and(t >= 0, t < PHASE_N)
                    dstbB[sl] = jnp.where(ok, t, PHASE_N)
                pltpu.async_copy(xi.at[srcbB], rowsB, semGA).wait()
                sa.wait()
                sb = pltpu.async_copy(rowsB, acc.at[dstbB], semSB, add=True)
                sb.wait()
                return 0

            lax.fori_loop(0, iters // 2, step, 0)
            plsc.subcore_barrier()

            out_rows = pl.ds(pl.multiple_of(pbase + s * OROWS, 8), OROWS)
            pltpu.sync_copy(acc.at[orows], agg_out.at[c, out_rows])
            plsc.subcore_barrier()
            return 0

        lax.fori_loop(0, NPHASES, phase_body, 0)

        if with_cnt:
            def cnt_phase(k, _):
                p = 2 * k + c

                @pl.when(p < NPHASES)
                def _():
                    pltpu.sync_copy(zbuf, acc.at[zrows])
                    plsc.subcore_barrier()
                    pbase = p * PHASE_N

                    def cstep(k, _):
                        pltpu.sync_copy(dstp.at[off1(2 * k)], dstbA)
                        remap_dst(dstbA, pbase)
                        sa = pltpu.async_copy(onesb, acc.at[dstbA], semSA,
                                              add=True)
                        pltpu.sync_copy(dstp.at[off1(2 * k + 1)], dstbB)
                        remap_dst(dstbB, pbase)
                        sb = pltpu.async_copy(onesb, acc.at[dstbB], semSB,
                                              add=True)
                        sa.wait()
                        sb.wait()
                        return 0

                    lax.fori_loop(0, iters // 2, cstep, 0)
                    plsc.subcore_barrier()
                    out_rows = pl.ds(
                        pl.multiple_of(pbase + s * OROWS, 8), OROWS)
                    pltpu.sync_copy(acc.at[orows], cnt_out.at[out_rows])
                    plsc.subcore_barrier()
                return 0

            lax.fori_loop(0, 2, cnt_phase, 0)

    return agg


def _tc_layer_body(a, cntr, xr, wl0, wl1, wr, br, o):
    rcp = 1.0 / jnp.maximum(cntr[:, 0:1], 1.0)
    dn = (((1,), (1,)), ((), ()))
    h = lax.dot_general(a[0] * rcp, wl0[...], dn,
                        preferred_element_type=jnp.float32)
    h = h + lax.dot_general(a[1] * rcp, wl1[...], dn,
                            preferred_element_type=jnp.float32)
    h = h + lax.dot_general(xr[...], wr[...], dn,
                            preferred_element_type=jnp.float32)
    o[...] = jnp.maximum(h + br[...], 0.0)


def _tc_final_body(a, cntr, xr, wl0, wl1, wr, br, wo, bo, o):
    rcp = 1.0 / jnp.maximum(cntr[:, 0:1], 1.0)
    dn = (((1,), (1,)), ((), ()))
    h = lax.dot_general(a[0] * rcp, wl0[...], dn,
                        preferred_element_type=jnp.float32)
    h = h + lax.dot_general(a[1] * rcp, wl1[...], dn,
                            preferred_element_type=jnp.float32)
    h = h + lax.dot_general(xr[...], wr[...], dn,
                            preferred_element_type=jnp.float32)
    h = jnp.maximum(h + br[...], 0.0)
    o[...] = lax.dot_general(h, wo[...], dn,
                             preferred_element_type=jnp.float32) + bo[...]


_ROW_BLK = 1000


def _tc_specs(out_cols):
    grid = (N // _ROW_BLK,)
    in_specs = [
        pl.BlockSpec((NC, _ROW_BLK, HALF), lambda i: (0, i, 0)),
        pl.BlockSpec((_ROW_BLK, HALF), lambda i: (i, 0)),
        pl.BlockSpec((_ROW_BLK, D), lambda i: (i, 0)),
        pl.BlockSpec((D, HALF), lambda i: (0, 0)),
        pl.BlockSpec((D, HALF), lambda i: (0, 0)),
        pl.BlockSpec((D, D), lambda i: (0, 0)),
        pl.BlockSpec((1, D), lambda i: (0, 0)),
    ]
    out_spec = pl.BlockSpec((_ROW_BLK, out_cols), lambda i: (i, 0))
    return grid, in_specs, out_spec


def _tc_layer(agg, cnt, xrows, W_l, b, W_r):
    grid, in_specs, out_spec = _tc_specs(D)
    return pl.pallas_call(
        _tc_layer_body,
        grid=grid,
        in_specs=in_specs,
        out_specs=out_spec,
        out_shape=jax.ShapeDtypeStruct((N, D), jnp.float32),
    )(agg, cnt, xrows, W_l[:, :HALF], W_l[:, HALF:], W_r, b.reshape(1, D))


def _tc_final(agg, cnt, xrows, W_l, b, W_r, W_out, b_out):
    grid, in_specs, out_spec = _tc_specs(OUT_D)
    in_specs = in_specs + [
        pl.BlockSpec((OUT_D, D), lambda i: (0, 0)),
        pl.BlockSpec((1, OUT_D), lambda i: (0, 0)),
    ]
    return pl.pallas_call(
        _tc_final_body,
        grid=grid,
        in_specs=in_specs,
        out_specs=out_spec,
        out_shape=jax.ShapeDtypeStruct((N, OUT_D), jnp.float32),
    )(agg, cnt, xrows, W_l[:, :HALF], W_l[:, HALF:], W_r, b.reshape(1, D),
      W_out, b_out.reshape(1, OUT_D))


def kernel(x, edge_index, W1_l, b1_l, W1_r, W2_l, b2_l, W2_r, W_out, b_out):
    e = edge_index.shape[1]
    e_pad = -(-e // (NS * CHUNK * 2)) * (NS * CHUNK * 2)
    src = edge_index[0].astype(jnp.int32)
    dst = edge_index[1].astype(jnp.int32)
    if e_pad != e:
        pad = e_pad - e
        src = jnp.concatenate([src, jnp.zeros((pad,), jnp.int32)])
        dst = jnp.concatenate([dst, jnp.full((pad,), TRASH_DST, jnp.int32)])

    agg_cnt = _make_agg_kernel(e_pad, True)
    agg_nocnt = _make_agg_kernel(e_pad, False)

    xi = x.reshape(N, 2, HALF).reshape(2 * N, HALF)
    a, cnt = agg_cnt(xi, src, dst)
    h1 = _tc_layer(a, cnt, x, W1_l, b1_l, W1_r)

    h1i = h1.reshape(N, 2, HALF).reshape(2 * N, HALF)
    agg2 = agg_nocnt(h1i, src, dst)
    if isinstance(agg2, (list, tuple)):
        agg2 = agg2[0]
    return _tc_final(agg2, cnt, h1, W2_l, b2_l, W2_r, W_out, b_out)


# speedup vs baseline: 1.3304x; 1.3304x over previous
import functools

import jax
import jax.numpy as jnp
from jax import lax
from jax.experimental import pallas as pl
from jax.experimental.pallas import tpu as pltpu
from jax.experimental.pallas import tpu_sc as plsc

N = 10000
D = 256
HALF = 128
OUT_D = 128

NC = 2
NS = 16
LANES = 16

CHUNK = 128
NPHASES = 3
PHASE_N = 3584
N_PAD = NPHASES * PHASE_N
ACC_ROWS = 3592
ZROWS = PHASE_N // NS
OROWS = PHASE_N // NS
TRASH_DST = N


def _make_agg_kernel(e_pad: int, with_cnt: bool):
    ept = e_pad // NS
    iters = ept // CHUNK

    out_type = [jax.ShapeDtypeStruct((NC, N_PAD, HALF), jnp.float32)]
    if with_cnt:
        out_type.append(jax.ShapeDtypeStruct((N_PAD, HALF), jnp.float32))

    scratch_types = [
        pltpu.VMEM_SHARED((ACC_ROWS, HALF), jnp.float32),
        pltpu.VMEM((ZROWS, HALF), jnp.float32),
        pltpu.VMEM((CHUNK, HALF), jnp.float32),
        pltpu.VMEM((2 * CHUNK,), jnp.int32),
        pltpu.VMEM((CHUNK,), jnp.int32),
        pltpu.VMEM((CHUNK,), jnp.int32),
        pltpu.VMEM((CHUNK, HALF), jnp.float32),
        pltpu.SemaphoreType.DMA,
    ]

    mesh = plsc.VectorSubcoreMesh(core_axis_name="c", subcore_axis_name="s")

    @functools.partial(
        pl.kernel, mesh=mesh, out_type=out_type, scratch_types=scratch_types)
    def agg(*refs):
        if with_cnt:
            (xi, sdp, agg_out, cnt_out,
             acc, zbuf, onesb, sdb, srcbA, dstbA, rowsA, semGA) = refs
        else:
            (xi, sdp, agg_out,
             acc, zbuf, onesb, sdb, srcbA, dstbA, rowsA, semGA) = refs
            cnt_out = None

        c = lax.axis_index("c")
        s = lax.axis_index("s")
        ebase = s * ept

        zero16 = jnp.zeros((LANES,), jnp.float32)
        one16 = jnp.ones((LANES,), jnp.float32)

        def zrow(i, _):
            for j in range(HALF // LANES):
                zbuf[i, pl.ds(j * LANES, LANES)] = zero16
            return 0

        lax.fori_loop(0, ZROWS, zrow, 0)

        if with_cnt:
            def orow(i, _):
                for j in range(HALF // LANES):
                    onesb[i, pl.ds(j * LANES, LANES)] = one16
                return 0
            lax.fori_loop(0, CHUNK, orow, 0)

        zrows = pl.ds(pl.multiple_of(s * ZROWS, 8), ZROWS)
        orows = pl.ds(pl.multiple_of(s * OROWS, 8), OROWS)

        def xform_src(buf):
            for j in range(CHUNK // LANES):
                sl = pl.ds(j * LANES, LANES)
                buf[sl] = buf[sl] * 2 + c

        def remap_dst(buf, pbase):
            for j in range(CHUNK // LANES):
                sl = pl.ds(j * LANES, LANES)
                t = buf[sl] - pbase
                ok = jnp.logical_and(t >= 0, t < PHASE_N)
                buf[sl] = jnp.where(ok, t, PHASE_N)

        def off1(i):
            return pl.ds(pl.multiple_of(ebase + i * CHUNK, CHUNK), CHUNK)

        def phase_body(phase, _):
            pltpu.sync_copy(zbuf, acc.at[zrows])
            plsc.subcore_barrier()

            pbase = phase * PHASE_N

            def step(i, _):
                off = pl.multiple_of(2 * (ebase + i * CHUNK), 2 * CHUNK)
                pltpu.sync_copy(sdp.at[pl.ds(off, 2 * CHUNK)], sdb)
                for j in range(CHUNK // LANES):
                    sl = pl.ds(j * LANES, LANES)
                    srcbA[sl] = sdb[sl] * 2 + c
                    t = sdb[pl.ds(CHUNK + j * LANES, LANES)] - pbase
                    ok = jnp.logical_and(t >= 0, t < PHASE_N)
                    dstbA[sl] = jnp.where(ok, t, PHASE_N)
                pltpu.async_copy(xi.at[srcbA], rowsA, semGA).wait()
                pltpu.sync_copy(rowsA, acc.at[dstbA], add=True)
                return 0

            lax.fori_loop(0, iters, step, 0)
            plsc.subcore_barrier()

            out_rows = pl.ds(pl.multiple_of(pbase + s * OROWS, 8), OROWS)
            pltpu.sync_copy(acc.at[orows], agg_out.at[c, out_rows])
            plsc.subcore_barrier()
            return 0

        lax.fori_loop(0, NPHASES, phase_body, 0)

        if with_cnt:
            def cnt_phase(k, _):
                p = 2 * k + c

                @pl.when(p < NPHASES)
                def _():
                    pltpu.sync_copy(zbuf, acc.at[zrows])
                    plsc.subcore_barrier()
                    pbase = p * PHASE_N

                    def cstep(i, _):
                        off = pl.multiple_of(
                            2 * (ebase + i * CHUNK) + CHUNK, CHUNK)
                        pltpu.sync_copy(sdp.at[pl.ds(off, CHUNK)], dstbA)
                        remap_dst(dstbA, pbase)
                        pltpu.sync_copy(onesb, acc.at[dstbA], add=True)
                        return 0

                    lax.fori_loop(0, iters, cstep, 0)
                    plsc.subcore_barrier()
                    out_rows = pl.ds(
                        pl.multiple_of(pbase + s * OROWS, 8), OROWS)
                    pltpu.sync_copy(acc.at[orows], cnt_out.at[out_rows])
                    plsc.subcore_barrier()
                return 0

            lax.fori_loop(0, 2, cnt_phase, 0)

    return agg


def _tc_layer_body(a, cntr, xr, wl0, wl1, wr, br, o):
    rcp = 1.0 / jnp.maximum(cntr[:, 0:1], 1.0)
    dn = (((1,), (1,)), ((), ()))
    h = lax.dot_general(a[0] * rcp, wl0[...], dn,
                        preferred_element_type=jnp.float32)
    h = h + lax.dot_general(a[1] * rcp, wl1[...], dn,
                            preferred_element_type=jnp.float32)
    h = h + lax.dot_general(xr[...], wr[...], dn,
                            preferred_element_type=jnp.float32)
    o[...] = jnp.maximum(h + br[...], 0.0)


def _tc_final_body(a, cntr, xr, wl0, wl1, wr, br, wo, bo, o):
    rcp = 1.0 / jnp.maximum(cntr[:, 0:1], 1.0)
    dn = (((1,), (1,)), ((), ()))
    h = lax.dot_general(a[0] * rcp, wl0[...], dn,
                        preferred_element_type=jnp.float32)
    h = h + lax.dot_general(a[1] * rcp, wl1[...], dn,
                            preferred_element_type=jnp.float32)
    h = h + lax.dot_general(xr[...], wr[...], dn,
                            preferred_element_type=jnp.float32)
    h = jnp.maximum(h + br[...], 0.0)
    o[...] = lax.dot_general(h, wo[...], dn,
                             preferred_element_type=jnp.float32) + bo[...]


_ROW_BLK = 1000


def _tc_specs(out_cols):
    grid = (N // _ROW_BLK,)
    in_specs = [
        pl.BlockSpec((NC, _ROW_BLK, HALF), lambda i: (0, i, 0)),
        pl.BlockSpec((_ROW_BLK, HALF), lambda i: (i, 0)),
        pl.BlockSpec((_ROW_BLK, D), lambda i: (i, 0)),
        pl.BlockSpec((D, HALF), lambda i: (0, 0)),
        pl.BlockSpec((D, HALF), lambda i: (0, 0)),
        pl.BlockSpec((D, D), lambda i: (0, 0)),
        pl.BlockSpec((1, D), lambda i: (0, 0)),
    ]
    out_spec = pl.BlockSpec((_ROW_BLK, out_cols), lambda i: (i, 0))
    return grid, in_specs, out_spec


def _tc_layer(agg, cnt, xrows, W_l, b, W_r):
    grid, in_specs, out_spec = _tc_specs(D)
    return pl.pallas_call(
        _tc_layer_body,
        grid=grid,
        in_specs=in_specs,
        out_specs=out_spec,
        out_shape=jax.ShapeDtypeStruct((N, D), jnp.float32),
    )(agg, cnt, xrows, W_l[:, :HALF], W_l[:, HALF:], W_r, b.reshape(1, D))


def _tc_final(agg, cnt, xrows, W_l, b, W_r, W_out, b_out):
    grid, in_specs, out_spec = _tc_specs(OUT_D)
    in_specs = in_specs + [
        pl.BlockSpec((OUT_D, D), lambda i: (0, 0)),
        pl.BlockSpec((1, OUT_D), lambda i: (0, 0)),
    ]
    return pl.pallas_call(
        _tc_final_body,
        grid=grid,
        in_specs=in_specs,
        out_specs=out_spec,
        out_shape=jax.ShapeDtypeStruct((N, OUT_D), jnp.float32),
    )(agg, cnt, xrows, W_l[:, :HALF], W_l[:, HALF:], W_r, b.reshape(1, D),
      W_out, b_out.reshape(1, OUT_D))


def kernel(x, edge_index, W1_l, b1_l, W1_r, W2_l, b2_l, W2_r, W_out, b_out):
    e = edge_index.shape[1]
    e_pad = -(-e // (NS * CHUNK)) * (NS * CHUNK)
    src = edge_index[0].astype(jnp.int32)
    dst = edge_index[1].astype(jnp.int32)
    if e_pad != e:
        pad = e_pad - e
        src = jnp.concatenate([src, jnp.zeros((pad,), jnp.int32)])
        dst = jnp.concatenate([dst, jnp.full((pad,), TRASH_DST, jnp.int32)])
    sd = jnp.concatenate(
        [src.reshape(-1, CHUNK), dst.reshape(-1, CHUNK)], axis=1).reshape(-1)

    agg_cnt = _make_agg_kernel(e_pad, True)
    agg_nocnt = _make_agg_kernel(e_pad, False)

    xi = x.reshape(N, 2, HALF).reshape(2 * N, HALF)
    a, cnt = agg_cnt(xi, sd)
    h1 = _tc_layer(a, cnt, x, W1_l, b1_l, W1_r)

    h1i = h1.reshape(N, 2, HALF).reshape(2 * N, HALF)
    agg2 = agg_nocnt(h1i, sd)
    if isinstance(agg2, (list, tuple)):
        agg2 = agg2[0]
    return _tc_final(agg2, cnt, h1, W2_l, b2_l, W2_r, W_out, b_out)
